# baseline (speedup 1.0000x reference)
import functools

import jax
import jax.numpy as jnp
from jax.experimental import pallas as pl
from jax.experimental.pallas import tpu as pltpu
from jax.experimental.pallas import tpu_sc as plsc

_SC_CORES = 2
_SC_SUBCORES = 16

_MID_BLK = 128
_DENSE_BLK = 200


def _sc_gather_rows(table, idx):
    rows = idx.shape[0]
    cols = table.shape[1]
    nw = _SC_CORES * _SC_SUBCORES
    b_per_w = rows // nw
    mesh = plsc.VectorSubcoreMesh(
        core_axis_name="c",
        subcore_axis_name="s",
        num_cores=_SC_CORES,
        num_subcores=_SC_SUBCORES,
    )

    @functools.partial(
        pl.kernel,
        out_type=jax.ShapeDtypeStruct((rows, cols), table.dtype),
        mesh=mesh,
        scratch_types=[
            pltpu.VMEM((b_per_w,), jnp.int32),
            pltpu.VMEM((b_per_w, cols), table.dtype),
            pltpu.SemaphoreType.DMA,
        ],
    )
    def gather(idx_hbm, table_hbm, out_hbm, idx_v, rows_v, sem):
        wid = jax.lax.axis_index("s") * _SC_CORES + jax.lax.axis_index("c")
        base = wid * b_per_w
        pltpu.sync_copy(idx_hbm.at[pl.ds(base, b_per_w)], idx_v)
        pltpu.async_copy(table_hbm.at[idx_v], rows_v, sem).wait()
        pltpu.sync_copy(rows_v, out_hbm.at[pl.ds(base, b_per_w)])

    return gather(idx, table)


def _fused_body(scale, s, n, idx_ref, g0_ref, w0_ref, b0_ref, w1_ref, b1_ref,
                a_hbm, a_blk, out_ref,
                acc0_ref, accb_ref, s0bf_ref, h0_ref, ag_ref, h_ref,
                scat1_ref, scat1b_ref, sem):
    i = pl.program_id(0)
    mid_steps = s // _MID_BLK
    dense0 = 1 + mid_steps

    def fire(g):
        buf = g % 2

        def body(j, _):
            r = idx_ref[s + g * _MID_BLK + j]
            pltpu.make_async_copy(
                a_hbm.at[pl.ds(r, 1), :],
                ag_ref.at[buf, pl.ds(j, 1), :],
                sem.at[buf],
            ).start()
            return 0

        jax.lax.fori_loop(0, _MID_BLK, body, 0, unroll=8)

    @pl.when(i == 0)
    def _():
        fire(0)
        h0 = jnp.dot(g0_ref[...], w0_ref[...], preferred_element_type=jnp.float32)
        h0_ref[...] = jnp.maximum(h0 + b0_ref[...], 0.0) * scale
        acc0_ref[...] = jnp.zeros_like(acc0_ref)
        accb_ref[...] = jnp.zeros_like(accb_ref)
        scat1_ref[...] = jnp.zeros_like(scat1_ref)
        scat1b_ref[...] = jnp.zeros_like(scat1b_ref)

        def body(j, _):
            ra = idx_ref[2 * j]
            acc0_ref[pl.ds(ra, 1), :] += h0_ref[pl.ds(2 * j, 1), :]
            rb = idx_ref[2 * j + 1]
            accb_ref[pl.ds(rb, 1), :] += h0_ref[pl.ds(2 * j + 1, 1), :]
            return 0

        jax.lax.fori_loop(0, s // 2, body, 0, unroll=4)
        s0bf_ref[...] = (acc0_ref[...] + accb_ref[...]).astype(jnp.bfloat16)

    @pl.when((i >= 1) & (i < dense0))
    def _():
        k = i - 1
        buf = k % 2

        @pl.when(k + 1 < mid_steps)
        def _():
            fire(k + 1)

        pltpu.make_async_copy(
            a_hbm.at[pl.ds(0, _MID_BLK), :], ag_ref.at[buf], sem.at[buf]
        ).wait()
        agg = jnp.dot(
            ag_ref[buf].astype(jnp.bfloat16),
            s0bf_ref[...],
            preferred_element_type=jnp.float32,
        )
        h1 = jnp.maximum(agg, 0.0)
        h1 = jnp.dot(h1, w1_ref[...], preferred_element_type=jnp.float32)
        h_ref[...] = jnp.maximum(h1 + b1_ref[...], 0.0) * scale

        def body(j, _):
            ra = idx_ref[s + k * _MID_BLK + 2 * j]
            scat1_ref[pl.ds(ra, 1), :] += h_ref[pl.ds(2 * j, 1), :]
            rb = idx_ref[s + k * _MID_BLK + 2 * j + 1]
            scat1b_ref[pl.ds(rb, 1), :] += h_ref[pl.ds(2 * j + 1, 1), :]
            return 0

        jax.lax.fori_loop(0, _MID_BLK // 2, body, 0, unroll=4)

    @pl.when(i >= dense0)
    def _():
        @pl.when(i == dense0)
        def _():
            scat1_ref[...] += scat1b_ref[...]

        out_ref[...] = jnp.dot(
            a_blk[...], scat1_ref[...], preferred_element_type=jnp.float32
        )


def kernel(X, sampled_nodes_per_layer, A_hat, W0, b0, W1, b1):
    n, din = X.shape
    s = sampled_nodes_per_layer.shape[1]
    dh = W0.shape[1]
    dout = W1.shape[1]
    scale = float(n) / float(s)
    s0 = sampled_nodes_per_layer[0]
    idx_flat = sampled_nodes_per_layer.reshape(2 * s)

    g0 = _sc_gather_rows(X, s0)

    mid_steps = s // _MID_BLK
    dense0 = 1 + mid_steps
    dense_steps = pl.cdiv(n, _DENSE_BLK)
    grid = dense0 + dense_steps

    def dense_map(i, idx_ref):
        return (jnp.where(i >= dense0, i - dense0, 0), 0)

    grid_spec = pltpu.PrefetchScalarGridSpec(
        num_scalar_prefetch=1,
        grid=(grid,),
        in_specs=[
            pl.BlockSpec((s, din), lambda i, idx_ref: (0, 0)),
            pl.BlockSpec((din, dh), lambda i, idx_ref: (0, 0)),
            pl.BlockSpec((1, dh), lambda i, idx_ref: (0, 0)),
            pl.BlockSpec((dh, dout), lambda i, idx_ref: (0, 0)),
            pl.BlockSpec((1, dout), lambda i, idx_ref: (0, 0)),
            pl.BlockSpec(memory_space=pltpu.MemorySpace.HBM),
            pl.BlockSpec((_DENSE_BLK, n), dense_map),
        ],
        out_specs=pl.BlockSpec((_DENSE_BLK, dout), dense_map),
        scratch_shapes=[
            pltpu.VMEM((n, dh), jnp.float32),
            pltpu.VMEM((n, dh), jnp.float32),
            pltpu.VMEM((n, dh), jnp.bfloat16),
            pltpu.VMEM((s, dh), jnp.float32),
            pltpu.VMEM((2, _MID_BLK, n), jnp.float32),
            pltpu.VMEM((_MID_BLK, dout), jnp.float32),
            pltpu.VMEM((n, dout), jnp.float32),
            pltpu.VMEM((n, dout), jnp.float32),
            pltpu.SemaphoreType.DMA((2,)),
        ],
    )
    return pl.pallas_call(
        functools.partial(_fused_body, scale, s, n),
        grid_spec=grid_spec,
        out_shape=jax.ShapeDtypeStruct((n, dout), jnp.float32),
    )(idx_flat, g0, W0, b0.reshape(1, dh), W1, b1.reshape(1, dout), A_hat, A_hat)


# speedup vs baseline: 1.0113x; 1.0113x over previous
import functools

import jax
import jax.numpy as jnp
from jax.experimental import pallas as pl
from jax.experimental.pallas import tpu as pltpu
from jax.experimental.pallas import tpu_sc as plsc

_SC_CORES = 2
_SC_SUBCORES = 16

_MID_BLK = 128
_DENSE_BLK = 200


def _sc_gather_rows(table, idx):
    rows = idx.shape[0]
    cols = table.shape[1]
    nw = _SC_CORES * _SC_SUBCORES
    b_per_w = rows // nw
    mesh = plsc.VectorSubcoreMesh(
        core_axis_name="c",
        subcore_axis_name="s",
        num_cores=_SC_CORES,
        num_subcores=_SC_SUBCORES,
    )

    @functools.partial(
        pl.kernel,
        out_type=jax.ShapeDtypeStruct((rows, cols), table.dtype),
        mesh=mesh,
        scratch_types=[
            pltpu.VMEM((b_per_w,), jnp.int32),
            pltpu.VMEM((b_per_w, cols), table.dtype),
            pltpu.SemaphoreType.DMA,
        ],
    )
    def gather(idx_hbm, table_hbm, out_hbm, idx_v, rows_v, sem):
        wid = jax.lax.axis_index("s") * _SC_CORES + jax.lax.axis_index("c")
        base = wid * b_per_w
        pltpu.sync_copy(idx_hbm.at[pl.ds(base, b_per_w)], idx_v)
        pltpu.async_copy(table_hbm.at[idx_v], rows_v, sem).wait()
        pltpu.sync_copy(rows_v, out_hbm.at[pl.ds(base, b_per_w)])

    return gather(idx, table)


def _fused_body(scale, s, n, idx_ref, g0_ref, w0_ref, b0_ref, w1_ref, b1_ref,
                a_hbm, a_blk, out_ref,
                acc0_ref, accb_ref, s0bf_ref, h0_ref, ag_ref, h_ref,
                scat1_ref, sem):
    i = pl.program_id(0)
    mid_steps = s // _MID_BLK
    dense0 = 1 + mid_steps

    def fire(g):
        buf = g % 3

        def body(j, _):
            r = idx_ref[s + g * _MID_BLK + j]
            pltpu.make_async_copy(
                a_hbm.at[pl.ds(r, 1), :],
                ag_ref.at[buf, pl.ds(j, 1), :],
                sem.at[buf],
            ).start()
            return 0

        jax.lax.fori_loop(0, _MID_BLK, body, 0, unroll=8)

    @pl.when(i == 0)
    def _():
        fire(0)
        fire(1)
        h0 = jnp.dot(g0_ref[...], w0_ref[...], preferred_element_type=jnp.float32)
        h0_ref[...] = jnp.maximum(h0 + b0_ref[...], 0.0) * scale
        acc0_ref[...] = jnp.zeros_like(acc0_ref)
        accb_ref[...] = jnp.zeros_like(accb_ref)
        scat1_ref[...] = jnp.zeros_like(scat1_ref)

        def body(j, _):
            ra = idx_ref[2 * j]
            acc0_ref[pl.ds(ra, 1), :] += h0_ref[pl.ds(2 * j, 1), :]
            rb = idx_ref[2 * j + 1]
            accb_ref[pl.ds(rb, 1), :] += h0_ref[pl.ds(2 * j + 1, 1), :]
            return 0

        jax.lax.fori_loop(0, s // 2, body, 0, unroll=4)
        s0bf_ref[...] = (acc0_ref[...] + accb_ref[...]).astype(jnp.bfloat16)

    @pl.when((i >= 1) & (i < dense0))
    def _():
        k = i - 1
        buf = k % 3
        pltpu.make_async_copy(
            a_hbm.at[pl.ds(0, _MID_BLK), :], ag_ref.at[buf], sem.at[buf]
        ).wait()
        agg = jnp.dot(
            ag_ref[buf].astype(jnp.bfloat16),
            s0bf_ref[...],
            preferred_element_type=jnp.float32,
        )
        h1 = jnp.maximum(agg, 0.0)
        h1 = jnp.dot(h1, w1_ref[...], preferred_element_type=jnp.float32)
        h_ref[...] = jnp.maximum(h1 + b1_ref[...], 0.0) * scale

        @pl.when(k + 2 < mid_steps)
        def _():
            fire(k + 2)

        def body(j, _):
            r = idx_ref[s + k * _MID_BLK + j]
            scat1_ref[pl.ds(r, 1), :] += h_ref[pl.ds(j, 1), :]
            return 0

        jax.lax.fori_loop(0, _MID_BLK, body, 0, unroll=8)

    @pl.when(i >= dense0)
    def _():
        out_ref[...] = jnp.dot(
            a_blk[...], scat1_ref[...], preferred_element_type=jnp.float32
        )


def kernel(X, sampled_nodes_per_layer, A_hat, W0, b0, W1, b1):
    n, din = X.shape
    s = sampled_nodes_per_layer.shape[1]
    dh = W0.shape[1]
    dout = W1.shape[1]
    scale = float(n) / float(s)
    s0 = sampled_nodes_per_layer[0]
    idx_flat = sampled_nodes_per_layer.reshape(2 * s)

    g0 = _sc_gather_rows(X, s0)

    mid_steps = s // _MID_BLK
    dense0 = 1 + mid_steps
    dense_steps = pl.cdiv(n, _DENSE_BLK)
    grid = dense0 + dense_steps

    def dense_map(i, idx_ref):
        return (jnp.where(i >= dense0, i - dense0, 0), 0)

    grid_spec = pltpu.PrefetchScalarGridSpec(
        num_scalar_prefetch=1,
        grid=(grid,),
        in_specs=[
            pl.BlockSpec((s, din), lambda i, idx_ref: (0, 0)),
            pl.BlockSpec((din, dh), lambda i, idx_ref: (0, 0)),
            pl.BlockSpec((1, dh), lambda i, idx_ref: (0, 0)),
            pl.BlockSpec((dh, dout), lambda i, idx_ref: (0, 0)),
            pl.BlockSpec((1, dout), lambda i, idx_ref: (0, 0)),
            pl.BlockSpec(memory_space=pltpu.MemorySpace.HBM),
            pl.BlockSpec((_DENSE_BLK, n), dense_map),
        ],
        out_specs=pl.BlockSpec((_DENSE_BLK, dout), dense_map),
        scratch_shapes=[
            pltpu.VMEM((n, dh), jnp.float32),
            pltpu.VMEM((n, dh), jnp.float32),
            pltpu.VMEM((n, dh), jnp.bfloat16),
            pltpu.VMEM((s, dh), jnp.float32),
            pltpu.VMEM((3, _MID_BLK, n), jnp.float32),
            pltpu.VMEM((_MID_BLK, dout), jnp.float32),
            pltpu.VMEM((n, dout), jnp.float32),
            pltpu.SemaphoreType.DMA((3,)),
        ],
    )
    return pl.pallas_call(
        functools.partial(_fused_body, scale, s, n),
        grid_spec=grid_spec,
        out_shape=jax.ShapeDtypeStruct((n, dout), jnp.float32),
    )(idx_flat, g0, W0, b0.reshape(1, dh), W1, b1.reshape(1, dout), A_hat, A_hat)


# speedup vs baseline: 1.0176x; 1.0062x over previous
import functools

import jax
import jax.numpy as jnp
from jax.experimental import pallas as pl
from jax.experimental.pallas import tpu as pltpu
from jax.experimental.pallas import tpu_sc as plsc

_SC_CORES = 2
_SC_SUBCORES = 16

_MID_BLK = 128
_DENSE_BLK = 256


def _sc_gather_rows(table, idx):
    rows = idx.shape[0]
    cols = table.shape[1]
    nw = _SC_CORES * _SC_SUBCORES
    b_per_w = rows // nw
    mesh = plsc.VectorSubcoreMesh(
        core_axis_name="c",
        subcore_axis_name="s",
        num_cores=_SC_CORES,
        num_subcores=_SC_SUBCORES,
    )

    @functools.partial(
        pl.kernel,
        out_type=jax.ShapeDtypeStruct((rows, cols), table.dtype),
        mesh=mesh,
        scratch_types=[
            pltpu.VMEM((b_per_w,), jnp.int32),
            pltpu.VMEM((b_per_w, cols), table.dtype),
            pltpu.SemaphoreType.DMA,
        ],
    )
    def gather(idx_hbm, table_hbm, out_hbm, idx_v, rows_v, sem):
        wid = jax.lax.axis_index("s") * _SC_CORES + jax.lax.axis_index("c")
        base = wid * b_per_w
        pltpu.sync_copy(idx_hbm.at[pl.ds(base, b_per_w)], idx_v)
        pltpu.async_copy(table_hbm.at[idx_v], rows_v, sem).wait()
        pltpu.sync_copy(rows_v, out_hbm.at[pl.ds(base, b_per_w)])

    return gather(idx, table)


def _fused_body(scale, s, n, idx_ref, g0_ref, w0_ref, b0_ref, w1_ref, b1_ref,
                a_hbm, a_blk, out_ref,
                acc0_ref, accb_ref, s0bf_ref, h0_ref, ag_ref, h_ref,
                scat1_ref, sem):
    i = pl.program_id(0)
    mid_steps = s // _MID_BLK
    dense0 = 1 + mid_steps

    def fire(g):
        buf = g % 3

        def body(j, _):
            r = idx_ref[s + g * _MID_BLK + j]
            pltpu.make_async_copy(
                a_hbm.at[pl.ds(r, 1), :],
                ag_ref.at[buf, pl.ds(j, 1), :],
                sem.at[buf],
            ).start()
            return 0

        jax.lax.fori_loop(0, _MID_BLK, body, 0, unroll=16)

    @pl.when(i == 0)
    def _():
        fire(0)
        fire(1)
        h0 = jnp.dot(g0_ref[...], w0_ref[...], preferred_element_type=jnp.float32)
        h0_ref[...] = jnp.maximum(h0 + b0_ref[...], 0.0) * scale
        acc0_ref[...] = jnp.zeros_like(acc0_ref)
        accb_ref[...] = jnp.zeros_like(accb_ref)
        scat1_ref[...] = jnp.zeros_like(scat1_ref)

        def body(j, _):
            ra = idx_ref[2 * j]
            acc0_ref[pl.ds(ra, 1), :] += h0_ref[pl.ds(2 * j, 1), :]
            rb = idx_ref[2 * j + 1]
            accb_ref[pl.ds(rb, 1), :] += h0_ref[pl.ds(2 * j + 1, 1), :]
            return 0

        jax.lax.fori_loop(0, s // 2, body, 0, unroll=4)
        s0bf_ref[...] = (acc0_ref[...] + accb_ref[...]).astype(jnp.bfloat16)

    @pl.when((i >= 1) & (i < dense0))
    def _():
        k = i - 1
        buf = k % 3
        pltpu.make_async_copy(
            a_hbm.at[pl.ds(0, _MID_BLK), :], ag_ref.at[buf], sem.at[buf]
        ).wait()
        agg = jnp.dot(
            ag_ref[buf].astype(jnp.bfloat16),
            s0bf_ref[...],
            preferred_element_type=jnp.float32,
        )
        h1 = jnp.maximum(agg, 0.0)
        h1 = jnp.dot(h1, w1_ref[...], preferred_element_type=jnp.float32)
        h_ref[...] = jnp.maximum(h1 + b1_ref[...], 0.0) * scale

        @pl.when(k + 2 < mid_steps)
        def _():
            fire(k + 2)

        def body(j, _):
            r = idx_ref[s + k * _MID_BLK + j]
            scat1_ref[pl.ds(r, 1), :] += h_ref[pl.ds(j, 1), :]
            return 0

        jax.lax.fori_loop(0, _MID_BLK, body, 0, unroll=8)

    @pl.when(i >= dense0)
    def _():
        out_ref[...] = jnp.dot(
            a_blk[...], scat1_ref[...], preferred_element_type=jnp.float32
        )


def kernel(X, sampled_nodes_per_layer, A_hat, W0, b0, W1, b1):
    n, din = X.shape
    s = sampled_nodes_per_layer.shape[1]
    dh = W0.shape[1]
    dout = W1.shape[1]
    scale = float(n) / float(s)
    s0 = sampled_nodes_per_layer[0]
    idx_flat = sampled_nodes_per_layer.reshape(2 * s)

    g0 = _sc_gather_rows(X, s0)

    mid_steps = s // _MID_BLK
    dense0 = 1 + mid_steps
    dense_steps = pl.cdiv(n, _DENSE_BLK)
    grid = dense0 + dense_steps

    def dense_map(i, idx_ref):
        return (jnp.where(i >= dense0, i - dense0, 0), 0)

    grid_spec = pltpu.PrefetchScalarGridSpec(
        num_scalar_prefetch=1,
        grid=(grid,),
        in_specs=[
            pl.BlockSpec((s, din), lambda i, idx_ref: (0, 0)),
            pl.BlockSpec((din, dh), lambda i, idx_ref: (0, 0)),
            pl.BlockSpec((1, dh), lambda i, idx_ref: (0, 0)),
            pl.BlockSpec((dh, dout), lambda i, idx_ref: (0, 0)),
            pl.BlockSpec((1, dout), lambda i, idx_ref: (0, 0)),
            pl.BlockSpec(memory_space=pltpu.MemorySpace.HBM),
            pl.BlockSpec((_DENSE_BLK, n), dense_map),
        ],
        out_specs=pl.BlockSpec((_DENSE_BLK, dout), dense_map),
        scratch_shapes=[
            pltpu.VMEM((n, dh), jnp.float32),
            pltpu.VMEM((n, dh), jnp.float32),
            pltpu.VMEM((n, dh), jnp.bfloat16),
            pltpu.VMEM((s, dh), jnp.float32),
            pltpu.VMEM((3, _MID_BLK, n), jnp.float32),
            pltpu.VMEM((_MID_BLK, dout), jnp.float32),
            pltpu.VMEM((n, dout), jnp.float32),
            pltpu.SemaphoreType.DMA((3,)),
        ],
    )
    return pl.pallas_call(
        functools.partial(_fused_body, scale, s, n),
        grid_spec=grid_spec,
        out_shape=jax.ShapeDtypeStruct((n, dout), jnp.float32),
    )(idx_flat, g0, W0, b0.reshape(1, dh), W1, b1.reshape(1, dout), A_hat, A_hat)


# speedup vs baseline: 1.0257x; 1.0080x over previous
import functools

import jax
import jax.numpy as jnp
from jax.experimental import pallas as pl
from jax.experimental.pallas import tpu as pltpu
from jax.experimental.pallas import tpu_sc as plsc

_SC_CORES = 2
_SC_SUBCORES = 16

_MID_BLK = 128
_DENSE_BLK = 256


def _sc_gather_rows(table, idx):
    rows = idx.shape[0]
    cols = table.shape[1]
    nw = _SC_CORES * _SC_SUBCORES
    b_per_w = rows // nw
    mesh = plsc.VectorSubcoreMesh(
        core_axis_name="c",
        subcore_axis_name="s",
        num_cores=_SC_CORES,
        num_subcores=_SC_SUBCORES,
    )

    @functools.partial(
        pl.kernel,
        out_type=jax.ShapeDtypeStruct((rows, cols), table.dtype),
        mesh=mesh,
        scratch_types=[
            pltpu.VMEM((b_per_w,), jnp.int32),
            pltpu.VMEM((b_per_w, cols), table.dtype),
            pltpu.SemaphoreType.DMA,
        ],
    )
    def gather(idx_hbm, table_hbm, out_hbm, idx_v, rows_v, sem):
        wid = jax.lax.axis_index("s") * _SC_CORES + jax.lax.axis_index("c")
        base = wid * b_per_w
        pltpu.sync_copy(idx_hbm.at[pl.ds(base, b_per_w)], idx_v)
        pltpu.async_copy(table_hbm.at[idx_v], rows_v, sem).wait()
        pltpu.sync_copy(rows_v, out_hbm.at[pl.ds(base, b_per_w)])

    return gather(idx, table)


def _fused_body(scale, s, n, idx_ref, g0_ref, w0_ref, b0_ref, w1_ref, b1_ref,
                a_hbm, a_blk, out_ref,
                acc0_ref, accb_ref, s0bf_ref, h0_ref, ag_ref, h_ref,
                scat1_ref, sem):
    i = pl.program_id(0)
    mid_steps = s // _MID_BLK
    dense0 = 1 + mid_steps

    def fire(g):
        buf = g % 3

        def body(j, _):
            r = idx_ref[s + g * _MID_BLK + j]
            pltpu.make_async_copy(
                a_hbm.at[pl.ds(r, 1), :],
                ag_ref.at[buf, pl.ds(j, 1), :],
                sem.at[buf],
            ).start()
            return 0

        jax.lax.fori_loop(0, _MID_BLK, body, 0, unroll=16)

    @pl.when(i == 0)
    def _():
        fire(0)
        fire(1)
        h0 = jnp.dot(g0_ref[...], w0_ref[...], preferred_element_type=jnp.float32)
        h0_ref[...] = jnp.maximum(h0 + b0_ref[...], 0.0) * scale
        acc0_ref[...] = jnp.zeros_like(acc0_ref)
        accb_ref[...] = jnp.zeros_like(accb_ref)
        scat1_ref[...] = jnp.zeros_like(scat1_ref)

        def body(j, _):
            ra = idx_ref[2 * j]
            acc0_ref[pl.ds(ra, 1), :] += h0_ref[pl.ds(2 * j, 1), :]
            rb = idx_ref[2 * j + 1]
            accb_ref[pl.ds(rb, 1), :] += h0_ref[pl.ds(2 * j + 1, 1), :]
            return 0

        jax.lax.fori_loop(0, s // 2, body, 0, unroll=8)
        s0bf_ref[...] = (acc0_ref[...] + accb_ref[...]).astype(jnp.bfloat16)

    @pl.when((i >= 1) & (i < dense0))
    def _():
        k = i - 1
        buf = k % 3
        pltpu.make_async_copy(
            a_hbm.at[pl.ds(0, _MID_BLK), :], ag_ref.at[buf], sem.at[buf]
        ).wait()
        agg = jnp.dot(
            ag_ref[buf].astype(jnp.bfloat16),
            s0bf_ref[...],
            preferred_element_type=jnp.float32,
        )
        h1 = jnp.maximum(agg, 0.0)
        h1 = jnp.dot(h1, w1_ref[...], preferred_element_type=jnp.float32)
        h_ref[...] = jnp.maximum(h1 + b1_ref[...], 0.0) * scale

        @pl.when(k + 2 < mid_steps)
        def _():
            fire(k + 2)

        def body(j, _):
            r = idx_ref[s + k * _MID_BLK + j]
            scat1_ref[pl.ds(r, 1), :] += h_ref[pl.ds(j, 1), :]
            return 0

        jax.lax.fori_loop(0, _MID_BLK, body, 0, unroll=16)

    @pl.when(i >= dense0)
    def _():
        out_ref[...] = jnp.dot(
            a_blk[...], scat1_ref[...], preferred_element_type=jnp.float32
        )


def kernel(X, sampled_nodes_per_layer, A_hat, W0, b0, W1, b1):
    n, din = X.shape
    s = sampled_nodes_per_layer.shape[1]
    dh = W0.shape[1]
    dout = W1.shape[1]
    scale = float(n) / float(s)
    s0 = sampled_nodes_per_layer[0]
    idx_flat = sampled_nodes_per_layer.reshape(2 * s)

    g0 = _sc_gather_rows(X, s0)

    mid_steps = s // _MID_BLK
    dense0 = 1 + mid_steps
    dense_steps = pl.cdiv(n, _DENSE_BLK)
    grid = dense0 + dense_steps

    def dense_map(i, idx_ref):
        return (jnp.where(i >= dense0, i - dense0, 0), 0)

    grid_spec = pltpu.PrefetchScalarGridSpec(
        num_scalar_prefetch=1,
        grid=(grid,),
        in_specs=[
            pl.BlockSpec((s, din), lambda i, idx_ref: (0, 0)),
            pl.BlockSpec((din, dh), lambda i, idx_ref: (0, 0)),
            pl.BlockSpec((1, dh), lambda i, idx_ref: (0, 0)),
            pl.BlockSpec((dh, dout), lambda i, idx_ref: (0, 0)),
            pl.BlockSpec((1, dout), lambda i, idx_ref: (0, 0)),
            pl.BlockSpec(memory_space=pltpu.MemorySpace.HBM),
            pl.BlockSpec((_DENSE_BLK, n), dense_map),
        ],
        out_specs=pl.BlockSpec((_DENSE_BLK, dout), dense_map),
        scratch_shapes=[
            pltpu.VMEM((n, dh), jnp.float32),
            pltpu.VMEM((n, dh), jnp.float32),
            pltpu.VMEM((n, dh), jnp.bfloat16),
            pltpu.VMEM((s, dh), jnp.float32),
            pltpu.VMEM((3, _MID_BLK, n), jnp.float32),
            pltpu.VMEM((_MID_BLK, dout), jnp.float32),
            pltpu.VMEM((n, dout), jnp.float32),
            pltpu.SemaphoreType.DMA((3,)),
        ],
    )
    return pl.pallas_call(
        functools.partial(_fused_body, scale, s, n),
        grid_spec=grid_spec,
        out_shape=jax.ShapeDtypeStruct((n, dout), jnp.float32),
    )(idx_flat, g0, W0, b0.reshape(1, dh), W1, b1.reshape(1, dout), A_hat, A_hat)
